# baseline (speedup 1.0000x reference)
import functools

import jax
import jax.numpy as jnp
from jax import lax
from jax.experimental import pallas as pl
from jax.experimental.pallas import tpu as pltpu
from jax.experimental.pallas import tpu_sc as plsc

_L = 16
_QW = 128


def _make_mask_kernel(batch, seq):
    cores, subs = 2, 16
    rows_per_core = batch // cores
    subs_per_row = subs // rows_per_core
    chunk = seq // subs_per_row
    nq = chunk // _QW

    mesh = plsc.VectorSubcoreMesh(core_axis_name="c", subcore_axis_name="s")

    @functools.partial(
        pl.kernel,
        mesh=mesh,
        out_type=jax.ShapeDtypeStruct((batch * seq,), jnp.float32),
        compiler_params=pltpu.CompilerParams(needs_layout_passes=False),
        scratch_types=[
            pltpu.VMEM((chunk,), jnp.int32),
            pltpu.VMEM((_L,), jnp.int32),
            pltpu.VMEM((nq, _QW), jnp.int32),
            pltpu.VMEM((nq, _QW), jnp.float32),
            pltpu.SemaphoreType.DMA,
        ],
    )
    def mask_kernel(orders_hbm, nb_hbm, out_hbm, raw_v, n_v, idx_v, vals_v, sem):
        c = lax.axis_index("c")
        s = lax.axis_index("s")
        b = c * rows_per_core + s // subs_per_row
        j0 = (s % subs_per_row) * chunk

        pltpu.sync_copy(orders_hbm.at[b, pl.ds(j0, chunk)], raw_v)
        pltpu.sync_copy(nb_hbm.at[b], n_v)
        nvec = n_v[...]
        jbase = lax.iota(jnp.int32, 16) + j0
        off = b * seq

        copies = []
        for q in range(nq):

            @plsc.parallel_loop(0, _QW, step=_L, unroll=8)
            def _body(r, q=q):
                t = q * _QW + r
                idx_v[q, pl.ds(r, _L)] = raw_v[pl.ds(t, _L)] + off
                vals_v[q, pl.ds(r, _L)] = jnp.where(
                    jbase + t < nvec, 1.0, 0.0
                )

            copies.append(
                pltpu.async_copy(vals_v.at[q], out_hbm.at[idx_v.at[q]], sem)
            )
        for cp in copies:
            cp.wait()

    return mask_kernel


def kernel(patches, orders):
    batch, seq, _ = patches.shape
    n = jax.random.randint(jax.random.key(42), (batch,), 1, seq + 1)
    nb = jnp.broadcast_to(n.astype(jnp.int32)[:, None], (batch, _L))
    idx = orders.astype(jnp.int32)
    flat = _make_mask_kernel(batch, seq)(idx, nb)
    return flat.reshape(batch, seq)


# speedup vs baseline: 2.4285x; 2.4285x over previous
import functools

import jax
import jax.numpy as jnp
from jax import lax
from jax.experimental import pallas as pl
from jax.experimental.pallas import tpu as pltpu
from jax.experimental.pallas import tpu_sc as plsc

_L = 16
_QW = 128


def _make_mask_kernel(batch, seq):
    cores, subs = 2, 16
    rows_per_core = batch // cores
    subs_per_row = subs // rows_per_core
    chunk = seq // subs_per_row
    nq = chunk // _QW

    mesh = plsc.VectorSubcoreMesh(core_axis_name="c", subcore_axis_name="s")

    @functools.partial(
        pl.kernel,
        mesh=mesh,
        out_type=jax.ShapeDtypeStruct((batch * seq,), jnp.float32),
        compiler_params=pltpu.CompilerParams(needs_layout_passes=False),
        scratch_types=[
            pltpu.VMEM((chunk,), jnp.int32),
            pltpu.VMEM((_L,), jnp.int32),
            pltpu.VMEM((nq, _QW), jnp.int32),
            pltpu.VMEM((nq, _QW), jnp.float32),
            pltpu.VMEM_SHARED((rows_per_core * seq,), jnp.float32),
            pltpu.SemaphoreType.DMA,
        ],
    )
    def mask_kernel(orders_hbm, nb_hbm, out_hbm, raw_v, n_v, idx_v, vals_v,
                    shared, sem):
        c = lax.axis_index("c")
        s = lax.axis_index("s")
        r_loc = s // subs_per_row
        b = c * rows_per_core + r_loc
        j0 = (s % subs_per_row) * chunk

        pltpu.sync_copy(orders_hbm.at[b, pl.ds(j0, chunk)], raw_v)
        pltpu.sync_copy(nb_hbm.at[b], n_v)
        nvec = n_v[...]
        jbase = lax.iota(jnp.int32, 16) + j0
        off = r_loc * seq

        copies = []
        for q in range(nq):

            @plsc.parallel_loop(0, _QW, step=_L, unroll=8)
            def _body(r, q=q):
                t = q * _QW + r
                idx_v[q, pl.ds(r, _L)] = raw_v[pl.ds(t, _L)] + off
                vals_v[q, pl.ds(r, _L)] = jnp.where(
                    jbase + t < nvec, 1.0, 0.0
                )

            copies.append(
                pltpu.async_copy(vals_v.at[q], shared.at[idx_v.at[q]], sem)
            )
        for cp in copies:
            cp.wait()
        plsc.subcore_barrier()

        @pl.when(s == 0)
        def _():
            pltpu.sync_copy(
                shared,
                out_hbm.at[pl.ds(c * rows_per_core * seq, rows_per_core * seq)],
            )

    return mask_kernel


def kernel(patches, orders):
    batch, seq, _ = patches.shape
    n = jax.random.randint(jax.random.key(42), (batch,), 1, seq + 1)
    nb = jnp.broadcast_to(n.astype(jnp.int32)[:, None], (batch, _L))
    idx = orders.astype(jnp.int32)
    flat = _make_mask_kernel(batch, seq)(idx, nb)
    return flat.reshape(batch, seq)


# speedup vs baseline: 2.6135x; 1.0762x over previous
import functools

import jax
import jax.numpy as jnp
from jax import lax
from jax.experimental import pallas as pl
from jax.experimental.pallas import tpu as pltpu
from jax.experimental.pallas import tpu_sc as plsc

_L = 16


def _make_mask_kernel(batch, seq):
    mesh = plsc.VectorSubcoreMesh(core_axis_name="c", subcore_axis_name="s")

    @functools.partial(
        pl.kernel,
        mesh=mesh,
        out_type=jax.ShapeDtypeStruct((batch, seq), jnp.float32),
        compiler_params=pltpu.CompilerParams(needs_layout_passes=False),
        scratch_types=[
            pltpu.VMEM((seq,), jnp.int32),
            pltpu.VMEM((_L,), jnp.int32),
            pltpu.VMEM((seq,), jnp.float32),
        ],
    )
    def mask_kernel(orders_hbm, nb_hbm, out_hbm, idx_v, n_v, row_v):
        wid = lax.axis_index("s") * 2 + lax.axis_index("c")

        @pl.when(wid < batch)
        def _():
            pltpu.sync_copy(orders_hbm.at[wid], idx_v)
            pltpu.sync_copy(nb_hbm.at[wid], n_v)
            nvec = n_v[...]
            jbase = lax.iota(jnp.int32, 16)

            @plsc.parallel_loop(0, seq, step=_L, unroll=16)
            def _body(j0):
                idx16 = idx_v[pl.ds(j0, _L)]
                vals = jnp.where(jbase + j0 < nvec, 1.0, 0.0)
                plsc.store_scatter(row_v, [idx16], vals)

            pltpu.sync_copy(row_v, out_hbm.at[wid])

    return mask_kernel


def kernel(patches, orders):
    batch, seq, _ = patches.shape
    n = jax.random.randint(jax.random.key(42), (batch,), 1, seq + 1)
    nb = jnp.broadcast_to(n.astype(jnp.int32)[:, None], (batch, _L))
    idx = orders.astype(jnp.int32)
    return _make_mask_kernel(batch, seq)(idx, nb)


# speedup vs baseline: 2.6576x; 1.0169x over previous
import functools

import jax
import jax.numpy as jnp
from jax import lax
from jax.experimental import pallas as pl
from jax.experimental.pallas import tpu as pltpu
from jax.experimental.pallas import tpu_sc as plsc

_L = 16


def _make_mask_kernel(batch, seq):
    mesh = plsc.VectorSubcoreMesh(core_axis_name="c", subcore_axis_name="s")

    @functools.partial(
        pl.kernel,
        mesh=mesh,
        out_type=jax.ShapeDtypeStruct((batch, seq), jnp.float32),
        compiler_params=pltpu.CompilerParams(needs_layout_passes=False),
        scratch_types=[
            pltpu.VMEM((seq,), jnp.int32),
            pltpu.VMEM((_L,), jnp.int32),
            pltpu.VMEM((seq,), jnp.float32),
            pltpu.SemaphoreType.DMA,
        ],
    )
    def mask_kernel(orders_hbm, nb_hbm, out_hbm, idx_v, n_v, row_v, sem):
        wid = lax.axis_index("s") * 2 + lax.axis_index("c")
        half = seq // 2

        @pl.when(wid < batch)
        def _():
            cp_n = pltpu.async_copy(nb_hbm.at[wid], n_v, sem)
            cp_a = pltpu.async_copy(
                orders_hbm.at[wid, pl.ds(0, half)], idx_v.at[pl.ds(0, half)], sem
            )
            cp_b = pltpu.async_copy(
                orders_hbm.at[wid, pl.ds(half, half)],
                idx_v.at[pl.ds(half, half)],
                sem,
            )
            cp_n.wait()
            cp_a.wait()
            nvec = n_v[...]
            jbase = lax.iota(jnp.int32, 16)

            @plsc.parallel_loop(0, half, step=_L, unroll=16)
            def _body_a(j0):
                idx16 = idx_v[pl.ds(j0, _L)]
                vals = jnp.where(jbase + j0 < nvec, 1.0, 0.0)
                plsc.store_scatter(row_v, [idx16], vals)

            cp_b.wait()

            @plsc.parallel_loop(half, seq, step=_L, unroll=16)
            def _body_b(j0):
                idx16 = idx_v[pl.ds(j0, _L)]
                vals = jnp.where(jbase + j0 < nvec, 1.0, 0.0)
                plsc.store_scatter(row_v, [idx16], vals)

            pltpu.sync_copy(row_v, out_hbm.at[wid])

    return mask_kernel


def kernel(patches, orders):
    batch, seq, _ = patches.shape
    n = jax.random.randint(jax.random.key(42), (batch,), 1, seq + 1)
    nb = jnp.broadcast_to(n.astype(jnp.int32)[:, None], (batch, _L))
    idx = orders.astype(jnp.int32)
    return _make_mask_kernel(batch, seq)(idx, nb)


# speedup vs baseline: 2.8814x; 1.0842x over previous
import functools

import jax
import jax.numpy as jnp
from jax import lax
from jax.experimental import pallas as pl
from jax.experimental.pallas import tpu as pltpu
from jax.experimental.pallas import tpu_sc as plsc

_L = 16


@functools.lru_cache(maxsize=None)
def _num_masked(batch, seq):
    import numpy as np

    with jax.ensure_compile_time_eval():
        n = jax.random.randint(jax.random.key(42), (batch,), 1, seq + 1)
        return tuple(int(x) for x in np.asarray(n))


@functools.lru_cache(maxsize=None)
def _make_mask_kernel(batch, seq):
    n_const = _num_masked(batch, seq)
    mesh = plsc.VectorSubcoreMesh(core_axis_name="c", subcore_axis_name="s")

    @functools.partial(
        pl.kernel,
        mesh=mesh,
        out_type=jax.ShapeDtypeStruct((batch, seq), jnp.float32),
        compiler_params=pltpu.CompilerParams(needs_layout_passes=False),
        scratch_types=[
            pltpu.VMEM((seq,), jnp.int32),
            pltpu.VMEM((seq,), jnp.float32),
            pltpu.SemaphoreType.DMA,
        ],
    )
    def mask_kernel(orders_hbm, out_hbm, idx_v, row_v, sem):
        wid = lax.axis_index("s") * 2 + lax.axis_index("c")
        half = seq // 2

        @pl.when(wid < batch)
        def _():
            cp_a = pltpu.async_copy(
                orders_hbm.at[wid, pl.ds(0, half)], idx_v.at[pl.ds(0, half)], sem
            )
            cp_b = pltpu.async_copy(
                orders_hbm.at[wid, pl.ds(half, half)],
                idx_v.at[pl.ds(half, half)],
                sem,
            )
            nw = jnp.int32(n_const[0])
            for b in range(1, batch):
                nw = jnp.where(wid == b, jnp.int32(n_const[b]), nw)
            jbase = lax.iota(jnp.int32, 16)

            cp_a.wait()

            @plsc.parallel_loop(0, half, step=_L, unroll=16)
            def _body_a(j0):
                idx16 = idx_v[pl.ds(j0, _L)]
                vals = jnp.where(jbase + j0 < nw, 1.0, 0.0)
                plsc.store_scatter(row_v, [idx16], vals)

            cp_b.wait()

            @plsc.parallel_loop(half, seq, step=_L, unroll=16)
            def _body_b(j0):
                idx16 = idx_v[pl.ds(j0, _L)]
                vals = jnp.where(jbase + j0 < nw, 1.0, 0.0)
                plsc.store_scatter(row_v, [idx16], vals)

            pltpu.sync_copy(row_v, out_hbm.at[wid])

    return mask_kernel


def kernel(patches, orders):
    batch, seq, _ = patches.shape
    idx = orders.astype(jnp.int32)
    return _make_mask_kernel(batch, seq)(idx)


# speedup vs baseline: 3.1458x; 1.0917x over previous
import functools

import jax
import jax.numpy as jnp
from jax import lax
from jax.experimental import pallas as pl
from jax.experimental.pallas import tpu as pltpu
from jax.experimental.pallas import tpu_sc as plsc

_L = 16


@functools.lru_cache(maxsize=None)
def _num_masked(batch, seq):
    import numpy as np

    with jax.ensure_compile_time_eval():
        n = jax.random.randint(jax.random.key(42), (batch,), 1, seq + 1)
        return tuple(int(x) for x in np.asarray(n))


@functools.lru_cache(maxsize=None)
def _make_mask_kernel(batch, seq):
    n_const = _num_masked(batch, seq)
    mesh = plsc.VectorSubcoreMesh(
        core_axis_name="c", subcore_axis_name="s", num_cores=1
    )

    @functools.partial(
        pl.kernel,
        mesh=mesh,
        out_type=jax.ShapeDtypeStruct((batch, seq), jnp.float32),
        compiler_params=pltpu.CompilerParams(needs_layout_passes=False),
        scratch_types=[
            pltpu.VMEM((seq,), jnp.int32),
            pltpu.VMEM((seq,), jnp.float32),
            pltpu.SemaphoreType.DMA,
        ],
    )
    def mask_kernel(orders_hbm, out_hbm, idx_v, row_v, sem):
        wid = lax.axis_index("s") + lax.axis_index("c")
        half = seq // 2

        @pl.when(wid < batch)
        def _():
            cp_a = pltpu.async_copy(
                orders_hbm.at[wid, pl.ds(0, half)], idx_v.at[pl.ds(0, half)], sem
            )
            cp_b = pltpu.async_copy(
                orders_hbm.at[wid, pl.ds(half, half)],
                idx_v.at[pl.ds(half, half)],
                sem,
            )
            nw = jnp.int32(n_const[0])
            for b in range(1, batch):
                nw = jnp.where(wid == b, jnp.int32(n_const[b]), nw)
            jbase = lax.iota(jnp.int32, 16)

            cp_a.wait()

            @plsc.parallel_loop(0, half, step=_L, unroll=16)
            def _body_a(j0):
                idx16 = idx_v[pl.ds(j0, _L)]
                vals = jnp.where(jbase + j0 < nw, 1.0, 0.0)
                plsc.store_scatter(row_v, [idx16], vals)

            cp_b.wait()

            @plsc.parallel_loop(half, seq, step=_L, unroll=16)
            def _body_b(j0):
                idx16 = idx_v[pl.ds(j0, _L)]
                vals = jnp.where(jbase + j0 < nw, 1.0, 0.0)
                plsc.store_scatter(row_v, [idx16], vals)

            pltpu.sync_copy(row_v, out_hbm.at[wid])

    return mask_kernel


def kernel(patches, orders):
    batch, seq, _ = patches.shape
    idx = orders.astype(jnp.int32)
    return _make_mask_kernel(batch, seq)(idx)
